# Initial kernel scaffold; baseline (speedup 1.0000x reference)
import jax
import jax.numpy as jnp
from jax.experimental import pallas as pl


def kernel(vocab_ids, table):
    raise NotImplementedError("write your pallas kernel here")


# speedup vs baseline: 6.0573x; 6.0573x over previous
import functools

import jax
import jax.numpy as jnp
from jax import lax
from jax.experimental import pallas as pl
from jax.experimental.pallas import tpu as pltpu
from jax.experimental.pallas import tpu_sc as plsc

_EMB = 64
_NUM_CORES = 2
_NUM_SUBCORES = 16
_NUM_WORKERS = _NUM_CORES * _NUM_SUBCORES
_CHUNK = 1024


@functools.lru_cache(maxsize=None)
def _make_gather(n_rows: int):
    rows_per_w = n_rows // _NUM_WORKERS
    n_chunks = rows_per_w // _CHUNK
    mesh = plsc.VectorSubcoreMesh(core_axis_name="c", subcore_axis_name="s")

    @functools.partial(
        pl.kernel,
        out_type=jax.ShapeDtypeStruct((n_rows, _EMB), jnp.float32),
        mesh=mesh,
        compiler_params=pltpu.CompilerParams(use_tc_tiling_on_sc=False),
        scratch_types=[
            pltpu.VMEM((_CHUNK,), jnp.int32),
            pltpu.VMEM((_CHUNK, _EMB), jnp.float32),
            pltpu.SemaphoreType.DMA,
        ],
    )
    def gather_kernel(ids_hbm, table_hbm, out_hbm, idx_v, rows_v, sem):
        wid = lax.axis_index("s") * _NUM_CORES + lax.axis_index("c")
        base = wid * rows_per_w

        def body(i, carry):
            off = base + i * _CHUNK
            pltpu.sync_copy(ids_hbm.at[pl.ds(off, _CHUNK)], idx_v)
            pltpu.async_copy(table_hbm.at[idx_v], rows_v, sem).wait()
            pltpu.sync_copy(rows_v, out_hbm.at[pl.ds(off, _CHUNK)])
            return carry

        lax.fori_loop(0, n_chunks, body, 0)

    return gather_kernel


def kernel(vocab_ids, table):
    b, s = vocab_ids.shape
    ids = vocab_ids.reshape(-1).astype(jnp.int32)
    out = _make_gather(b * s)(ids, table)
    return out.reshape(b, s, _EMB)


# speedup vs baseline: 6.0891x; 1.0052x over previous
import functools

import jax
import jax.numpy as jnp
from jax import lax
from jax.experimental import pallas as pl
from jax.experimental.pallas import tpu as pltpu
from jax.experimental.pallas import tpu_sc as plsc

_EMB = 64
_NUM_CORES = 2
_NUM_SUBCORES = 16
_NUM_WORKERS = _NUM_CORES * _NUM_SUBCORES
_CHUNK = 512


@functools.lru_cache(maxsize=None)
def _make_gather(n_rows: int):
    rows_per_w = n_rows // _NUM_WORKERS
    n_chunks = rows_per_w // _CHUNK
    n_pairs = n_chunks // 2
    mesh = plsc.VectorSubcoreMesh(core_axis_name="c", subcore_axis_name="s")

    @functools.partial(
        pl.kernel,
        out_type=jax.ShapeDtypeStruct((n_rows, _EMB), jnp.float32),
        mesh=mesh,
        compiler_params=pltpu.CompilerParams(use_tc_tiling_on_sc=False),
        scratch_types=[
            pltpu.VMEM((_CHUNK,), jnp.int32),
            pltpu.VMEM((_CHUNK,), jnp.int32),
            pltpu.VMEM((_CHUNK, _EMB), jnp.float32),
            pltpu.VMEM((_CHUNK, _EMB), jnp.float32),
            pltpu.SemaphoreType.DMA,
            pltpu.SemaphoreType.DMA,
            pltpu.SemaphoreType.DMA,
            pltpu.SemaphoreType.DMA,
        ],
    )
    def gather_kernel(ids_hbm, table_hbm, out_hbm,
                      idx0, idx1, rows0, rows1, gsem0, gsem1, ssem0, ssem1):
        wid = lax.axis_index("s") * _NUM_CORES + lax.axis_index("c")
        base = wid * rows_per_w

        def step(j, i, idx_v, rows_v, gsem, ssem):
            off = base + i * _CHUNK
            @pl.when(j >= 1)
            def _():
                prev = off - 2 * _CHUNK
                pltpu.make_async_copy(
                    rows_v, out_hbm.at[pl.ds(prev, _CHUNK)], ssem).wait()

            pltpu.sync_copy(ids_hbm.at[pl.ds(off, _CHUNK)], idx_v)
            pltpu.async_copy(table_hbm.at[idx_v], rows_v, gsem).wait()
            pltpu.async_copy(rows_v, out_hbm.at[pl.ds(off, _CHUNK)], ssem)

        def body(j, carry):
            step(j, 2 * j, idx0, rows0, gsem0, ssem0)
            step(j, 2 * j + 1, idx1, rows1, gsem1, ssem1)
            return carry

        lax.fori_loop(0, n_pairs, body, 0)

        last0 = base + (n_chunks - 2) * _CHUNK
        last1 = base + (n_chunks - 1) * _CHUNK
        pltpu.make_async_copy(
            rows0, out_hbm.at[pl.ds(last0, _CHUNK)], ssem0).wait()
        pltpu.make_async_copy(
            rows1, out_hbm.at[pl.ds(last1, _CHUNK)], ssem1).wait()

    return gather_kernel


def kernel(vocab_ids, table):
    b, s = vocab_ids.shape
    ids = vocab_ids.reshape(-1).astype(jnp.int32)
    out = _make_gather(b * s)(ids, table)
    return out.reshape(b, s, _EMB)


# speedup vs baseline: 6.2214x; 1.0217x over previous
import functools

import jax
import jax.numpy as jnp
from jax import lax
from jax.experimental import pallas as pl
from jax.experimental.pallas import tpu as pltpu
from jax.experimental.pallas import tpu_sc as plsc

_EMB = 64
_NUM_CORES = 2
_NUM_SUBCORES = 16
_NUM_WORKERS = _NUM_CORES * _NUM_SUBCORES
_CHUNK = 256
_NBUF = 4


@functools.lru_cache(maxsize=None)
def _make_gather(n_rows: int):
    rows_per_w = n_rows // _NUM_WORKERS
    n_chunks = rows_per_w // _CHUNK
    n_groups = n_chunks // _NBUF
    mesh = plsc.VectorSubcoreMesh(core_axis_name="c", subcore_axis_name="s")

    @functools.partial(
        pl.kernel,
        out_type=jax.ShapeDtypeStruct((n_rows, _EMB), jnp.float32),
        mesh=mesh,
        compiler_params=pltpu.CompilerParams(use_tc_tiling_on_sc=False),
        scratch_types=[
            [pltpu.VMEM((_CHUNK,), jnp.int32) for _ in range(_NBUF)],
            [pltpu.VMEM((_CHUNK, _EMB), jnp.float32) for _ in range(_NBUF)],
            [pltpu.SemaphoreType.DMA for _ in range(_NBUF)],
            [pltpu.SemaphoreType.DMA for _ in range(_NBUF)],
        ],
    )
    def gather_kernel(ids_hbm, table_hbm, out_hbm, idx, rows, gsem, ssem):
        wid = lax.axis_index("s") * _NUM_CORES + lax.axis_index("c")
        base = wid * rows_per_w

        def body(j, carry):
            goff = base + j * _NBUF * _CHUNK
            for b in range(_NBUF):
                off = goff + b * _CHUNK

                @pl.when(j >= 1)
                def _(off=off, b=b):
                    prev = off - _NBUF * _CHUNK
                    pltpu.make_async_copy(
                        rows[b], out_hbm.at[pl.ds(prev, _CHUNK)], ssem[b]
                    ).wait()

                pltpu.sync_copy(ids_hbm.at[pl.ds(off, _CHUNK)], idx[b])
                pltpu.async_copy(table_hbm.at[idx[b]], rows[b], gsem[b])
            for b in range(_NBUF):
                off = goff + b * _CHUNK
                pltpu.make_async_copy(
                    table_hbm.at[idx[b]], rows[b], gsem[b]).wait()
                pltpu.async_copy(rows[b], out_hbm.at[pl.ds(off, _CHUNK)], ssem[b])
            return carry

        lax.fori_loop(0, n_groups, body, 0)

        for b in range(_NBUF):
            last = base + ((n_groups - 1) * _NBUF + b) * _CHUNK
            pltpu.make_async_copy(
                rows[b], out_hbm.at[pl.ds(last, _CHUNK)], ssem[b]).wait()

    return gather_kernel


def kernel(vocab_ids, table):
    b, s = vocab_ids.shape
    ids = vocab_ids.reshape(-1).astype(jnp.int32)
    out = _make_gather(b * s)(ids, table)
    return out.reshape(b, s, _EMB)
